# Initial kernel scaffold; baseline (speedup 1.0000x reference)
import jax
import jax.numpy as jnp
from jax.experimental import pallas as pl


def kernel(x, edge_index, edge_attr, g0Wn, g0We, g0b, u0Wn, u0We, u0b, g1Wn, g1We, g1b, u1Wn, u1We, u1b):
    raise NotImplementedError("write your pallas kernel here")


# speedup vs baseline: 7.3491x; 7.3491x over previous
import functools

import jax
import jax.numpy as jnp
from jax import lax
from jax.experimental import pallas as pl
from jax.experimental.pallas import tpu as pltpu
from jax.experimental.pallas import tpu_sc as plsc

N = 10000
T = 8
E = 160000
F_EDGE = 16
HID = 128
WC = 128
NCH = 2
NROWS = 640

NC = 2
NS = 16

_E_PER_TILE = E // NS
_KCH = 80
_NCHUNK = _E_PER_TILE // _KCH
_ROWS_PER_TILE = NROWS // NS
NPAD = 10240
_N_PER_TILE = NPAD // NS
_ZROWS = 64


def _rsqrt16(x):
    i = plsc.bitcast(x, jnp.int32)
    i = jnp.int32(0x5F3759DF) - lax.shift_right_arithmetic(i, jnp.int32(1))
    y = plsc.bitcast(i, jnp.float32)
    for _ in range(3):
        y = y * (1.5 - 0.5 * x * y * y)
    return y


def _k1_body(src_hbm, dst_hbm, isq_hbm, isqsrc_hbm,
             acc_sp, isq_sp, dstv, obuf, idxbuf, tmp, isq_tab, outv, sem):
    cid = lax.axis_index("c")
    s = lax.axis_index("s")

    @pl.when(cid == 0)
    def _():
        base = s * _E_PER_TILE
        pltpu.sync_copy(dst_hbm.at[pl.ds(base, _E_PER_TILE)], dstv)

        @functools.partial(lax.fori_loop, 0, _ROWS_PER_TILE, init_val=None)
        def _(i, _):
            tmp[i, :] = jnp.zeros((16,), jnp.float32)
            return None

        pltpu.sync_copy(tmp, acc_sp.at[pl.ds(s * _ROWS_PER_TILE,
                                             _ROWS_PER_TILE)])
        plsc.subcore_barrier()

        lane_iota = lax.iota(jnp.int32, 16)

        @functools.partial(lax.fori_loop, 0, _NCHUNK, init_val=None)
        def _(j, _):
            coff = j * _KCH

            @functools.partial(lax.fori_loop, 0, _KCH // 16, init_val=None)
            def _(i, _):
                d = dstv[pl.ds(coff + i * 16, 16)]
                idxbuf[pl.ds(i * 16, 16)] = lax.shift_right_logical(d, 4)
                for k in range(16):
                    dk = d[k]
                    oh = jnp.where(lane_iota == jnp.bitwise_and(dk, 15),
                                   1.0, 0.0).astype(jnp.float32)
                    obuf[i * 16 + k, :] = oh
                return None

            pltpu.sync_copy(obuf, acc_sp.at[idxbuf], add=True)
            return None

        plsc.subcore_barrier()

        rbase = s * _ROWS_PER_TILE
        pltpu.sync_copy(acc_sp.at[pl.ds(rbase, _ROWS_PER_TILE)], tmp)

        @functools.partial(lax.fori_loop, 0, _ROWS_PER_TILE, init_val=None)
        def _(i, _):
            outv[pl.ds(i * 16, 16)] = _rsqrt16(tmp[i, :] + 1.0)
            return None

        flat = pl.ds(rbase * 16, _ROWS_PER_TILE * 16)
        pltpu.sync_copy(outv.at[pl.ds(0, _ROWS_PER_TILE * 16)],
                        isq_sp.at[flat])
        pltpu.sync_copy(outv.at[pl.ds(0, _ROWS_PER_TILE * 16)],
                        isq_hbm.at[flat])
        plsc.subcore_barrier()

        pltpu.sync_copy(isq_sp, isq_tab)
        pltpu.sync_copy(src_hbm.at[pl.ds(base, _E_PER_TILE)], dstv)

        @functools.partial(lax.fori_loop, 0, _E_PER_TILE // 16, init_val=None)
        def _(i, _):
            sidx = dstv[pl.ds(i * 16, 16)]
            outv[pl.ds(i * 16, 16)] = plsc.load_gather(isq_tab, [sidx])
            return None

        pltpu.sync_copy(outv, isqsrc_hbm.at[pl.ds(base, _E_PER_TILE)])


def _compute_isq(src, dst):
    return pl.kernel(
        _k1_body,
        out_type=(jax.ShapeDtypeStruct((NROWS * 16,), jnp.float32),
                  jax.ShapeDtypeStruct((E,), jnp.float32)),
        mesh=plsc.VectorSubcoreMesh(core_axis_name="c", subcore_axis_name="s"),
        scratch_types=(
            pltpu.VMEM_SHARED((NROWS, 16), jnp.float32),
            pltpu.VMEM_SHARED((NROWS * 16,), jnp.float32),
            pltpu.VMEM((_E_PER_TILE,), jnp.int32),
            pltpu.VMEM((_KCH, 16), jnp.float32),
            pltpu.VMEM((_KCH,), jnp.int32),
            pltpu.VMEM((_ROWS_PER_TILE, 16), jnp.float32),
            pltpu.VMEM((NROWS * 16,), jnp.float32),
            pltpu.VMEM((_E_PER_TILE,), jnp.float32),
            pltpu.SemaphoreType.DMA,
        ),
        compiler_params=pltpu.CompilerParams(needs_layout_passes=False,
                                             use_tc_tiling_on_sc=False),
        name="sc_deg_isq",
    )(src, dst)


_N_ITEMS = T * NCH + NCH


def _k2_body(hs_hbm, es_hbm, src_hbm, dst_hbm, s_hbm, p_hbm,
             acc_sp, srcv, dstv, gbuf, zbuf, sem):
    cid = lax.axis_index("c")
    s = lax.axis_index("s")
    ebase = s * _E_PER_TILE

    pltpu.sync_copy(src_hbm.at[pl.ds(ebase, _E_PER_TILE)], srcv)
    for j in range(_NCHUNK):
        pltpu.sync_copy(dst_hbm.at[pl.ds(ebase + j * _KCH, _KCH)], dstv.at[j])

    @functools.partial(lax.fori_loop, 0, _ZROWS, init_val=None)
    def _(i, _):
        for l in range(WC // 16):
            zbuf[i, pl.ds(l * 16, 16)] = jnp.zeros((16,), jnp.float32)
        return None

    def zero_my_slice():
        for z in range(_N_PER_TILE // _ZROWS):
            pltpu.sync_copy(
                zbuf, acc_sp.at[pl.ds(s * _N_PER_TILE + z * _ZROWS, _ZROWS)])

    zero_my_slice()
    plsc.subcore_barrier()

    for g in range(_N_ITEMS):
        my = cid == (g % NC)

        @pl.when(my)
        def _(g=g):
            if g < T * NCH:
                table = hs_hbm.at[g]

                @functools.partial(lax.fori_loop, 0, _NCHUNK, init_val=None)
                def _(j, _):
                    pltpu.async_copy(
                        table.at[srcv.at[pl.ds(j * _KCH, _KCH)]],
                        gbuf, sem).wait()
                    pltpu.sync_copy(gbuf, acc_sp.at[dstv.at[j]], add=True)
                    return None
            else:
                table = es_hbm.at[g - T * NCH]

                @functools.partial(lax.fori_loop, 0, _NCHUNK, init_val=None)
                def _(j, _):
                    pltpu.async_copy(
                        table.at[pl.ds(ebase + j * _KCH, _KCH)],
                        gbuf, sem).wait()
                    pltpu.sync_copy(gbuf, acc_sp.at[dstv.at[j]], add=True)
                    return None

        plsc.subcore_barrier()

        @pl.when(my)
        def _(g=g):
            nslice = pl.ds(s * _N_PER_TILE, _N_PER_TILE)
            if g < T * NCH:
                pltpu.sync_copy(acc_sp.at[nslice], s_hbm.at[g].at[nslice])
            else:
                pltpu.sync_copy(acc_sp.at[nslice],
                                p_hbm.at[g - T * NCH].at[nslice])
            zero_my_slice()

        plsc.subcore_barrier()


def _segment_sums(hs_items, es, src, dst):
    return pl.kernel(
        _k2_body,
        out_type=(jax.ShapeDtypeStruct((T * NCH, NPAD, WC), jnp.float32),
                  jax.ShapeDtypeStruct((NCH, NPAD, WC), jnp.float32)),
        mesh=plsc.VectorSubcoreMesh(core_axis_name="c", subcore_axis_name="s"),
        scratch_types=(
            pltpu.VMEM_SHARED((NPAD, WC), jnp.float32),
            pltpu.VMEM((_E_PER_TILE,), jnp.int32),
            pltpu.VMEM((_NCHUNK, _KCH), jnp.int32),
            pltpu.VMEM((_KCH, WC), jnp.float32),
            pltpu.VMEM((_ZROWS, WC), jnp.float32),
            pltpu.SemaphoreType.DMA,
        ),
        compiler_params=pltpu.CompilerParams(needs_layout_passes=False,
                                             use_tc_tiling_on_sc=False),
        name="sc_segment_sums",
    )(hs_items, es, src, dst)


_NT = 1000
_ET = 2000


def _k3a_body(x_ref, w_ref, isq_ref, hs_ref):
    y = jnp.dot(x_ref[0], w_ref[...], preferred_element_type=jnp.float32)
    y = y * isq_ref[...]
    for ci in range(NCH):
        hs_ref[0, ci] = y[:, ci * WC:(ci + 1) * WC]


def _node_proj(x, wcat, isq_col):
    return pl.pallas_call(
        _k3a_body,
        grid=(T, N // _NT),
        in_specs=[
            pl.BlockSpec((1, _NT, HID), lambda t, n: (t, n, 0)),
            pl.BlockSpec((HID, 2 * HID), lambda t, n: (0, 0)),
            pl.BlockSpec((_NT, 1), lambda t, n: (n, 0)),
        ],
        out_specs=pl.BlockSpec((1, NCH, _NT, WC), lambda t, n: (t, 0, n, 0)),
        out_shape=jax.ShapeDtypeStruct((T, NCH, N, WC), jnp.float32),
    )(x, wcat, isq_col)


def _k3b_body(ea_ref, w_ref, isqs_ref, es_ref):
    y = jnp.dot(ea_ref[...], w_ref[...], preferred_element_type=jnp.float32)
    y = jnp.maximum(y, 0.0) * isqs_ref[...]
    for ci in range(NCH):
        es_ref[ci] = y[:, ci * WC:(ci + 1) * WC]


def _edge_proj(edge_attr, wecat, isq_src_col):
    return pl.pallas_call(
        _k3b_body,
        grid=(E // _ET,),
        in_specs=[
            pl.BlockSpec((_ET, F_EDGE), lambda e: (e, 0)),
            pl.BlockSpec((F_EDGE, 2 * HID), lambda e: (0, 0)),
            pl.BlockSpec((_ET, 1), lambda e: (e, 0)),
        ],
        out_specs=pl.BlockSpec((NCH, _ET, WC), lambda e: (0, e, 0)),
        out_shape=jax.ShapeDtypeStruct((NCH, E, WC), jnp.float32),
    )(edge_attr, wecat, isq_src_col)


def _k5_body(s_ref, p_ref, hs_ref, isq_ref, b_ref, out_ref, *, apply_relu):
    isq = isq_ref[...]
    g0 = isq * (s_ref[0, 0] + p_ref[0] + hs_ref[0, 0]) + b_ref[0:1, 0:WC]
    g1 = isq * (s_ref[0, 1] + p_ref[1] + hs_ref[0, 1]) + b_ref[0:1, WC:2 * WC]
    o = (1.0 - jax.nn.sigmoid(g0)) * jnp.tanh(g1)
    if apply_relu:
        o = jnp.maximum(o, 0.0)
    out_ref[0] = o


def _gate_combine(S4, P, hs, isq_col, bcat, apply_relu):
    return pl.pallas_call(
        functools.partial(_k5_body, apply_relu=apply_relu),
        grid=(T, N // _NT),
        in_specs=[
            pl.BlockSpec((1, NCH, _NT, WC), lambda t, n: (t, 0, n, 0)),
            pl.BlockSpec((NCH, _NT, WC), lambda t, n: (0, n, 0)),
            pl.BlockSpec((1, NCH, _NT, WC), lambda t, n: (t, 0, n, 0)),
            pl.BlockSpec((_NT, 1), lambda t, n: (n, 0)),
            pl.BlockSpec((1, 2 * HID), lambda t, n: (0, 0)),
        ],
        out_specs=pl.BlockSpec((1, _NT, HID), lambda t, n: (t, n, 0)),
        out_shape=jax.ShapeDtypeStruct((T, N, HID), jnp.float32),
    )(S4, P, hs, isq_col, bcat)


def _layer(x_in, src, dst, edge_attr, isq_col, isq_src_col,
           gWn, gWe, gb, uWn, uWe, ub, apply_relu):
    f = x_in.shape[-1]
    wcat = jnp.concatenate([gWn[:f, HID:], uWn[:f, :]], axis=1)
    wecat = jnp.concatenate([gWe[:, HID:], uWe], axis=1)
    bcat = jnp.concatenate([gb[HID:], ub]).reshape(1, 2 * HID)

    hs = _node_proj(x_in, wcat, isq_col)
    es = _edge_proj(edge_attr, wecat, isq_src_col)
    S, P = _segment_sums(hs.reshape(T * NCH, N, WC), es, src, dst)
    return _gate_combine(S.reshape(T, NCH, NPAD, WC), P, hs, isq_col, bcat,
                         apply_relu)


def kernel(x, edge_index, edge_attr, g0Wn, g0We, g0b, u0Wn, u0We, u0b,
           g1Wn, g1We, g1b, u1Wn, u1We, u1b):
    src = edge_index[0]
    dst = edge_index[1]

    isq_flat, isq_src = _compute_isq(src, dst)
    isq_col = isq_flat[:N].reshape(N, 1)
    isq_src_col = isq_src.reshape(E, 1)

    h0 = _layer(x, src, dst, edge_attr, isq_col, isq_src_col,
                g0Wn, g0We, g0b, u0Wn, u0We, u0b, apply_relu=True)
    h1 = _layer(h0, src, dst, edge_attr, isq_col, isq_src_col,
                g1Wn, g1We, g1b, u1Wn, u1We, u1b, apply_relu=False)
    return h1


# speedup vs baseline: 11.5817x; 1.5759x over previous
import functools

import jax
import jax.numpy as jnp
from jax import lax
from jax.experimental import pallas as pl
from jax.experimental.pallas import tpu as pltpu
from jax.experimental.pallas import tpu_sc as plsc

N = 10000
T = 8
E = 160000
F_EDGE = 16
HID = 128
WC = 128
NCH = 2
NROWS = 640

NC = 2
NS = 16

_E_PER_TILE = E // NS
_KCH = 80
_NCHUNK = _E_PER_TILE // _KCH
_ROWS_PER_TILE = NROWS // NS
NPAD = 10240
_N_PER_TILE = NPAD // NS
_ZROWS = 32


def _rsqrt16(x):
    i = plsc.bitcast(x, jnp.int32)
    i = jnp.int32(0x5F3759DF) - lax.shift_right_arithmetic(i, jnp.int32(1))
    y = plsc.bitcast(i, jnp.float32)
    for _ in range(3):
        y = y * (1.5 - 0.5 * x * y * y)
    return y


def _k1_body(src_hbm, dst_hbm, isq_hbm, isqsrc_hbm,
             acc_sp, isq_sp, dstv, obuf, idxbuf, tmp, isq_tab, outv, sem):
    cid = lax.axis_index("c")
    s = lax.axis_index("s")

    @pl.when(cid == 0)
    def _():
        base = s * _E_PER_TILE
        pltpu.sync_copy(dst_hbm.at[pl.ds(base, _E_PER_TILE)], dstv)

        @functools.partial(lax.fori_loop, 0, _ROWS_PER_TILE, init_val=None)
        def _(i, _):
            tmp[i, :] = jnp.zeros((16,), jnp.float32)
            return None

        pltpu.sync_copy(tmp, acc_sp.at[pl.ds(s * _ROWS_PER_TILE,
                                             _ROWS_PER_TILE)])
        plsc.subcore_barrier()

        lane_iota = lax.iota(jnp.int32, 16)

        @functools.partial(lax.fori_loop, 0, _NCHUNK, init_val=None)
        def _(j, _):
            coff = j * _KCH

            @functools.partial(lax.fori_loop, 0, _KCH // 16, init_val=None)
            def _(i, _):
                d = dstv[pl.ds(coff + i * 16, 16)]
                idxbuf[pl.ds(i * 16, 16)] = lax.shift_right_logical(d, 4)
                for k in range(16):
                    dk = d[k]
                    oh = jnp.where(lane_iota == jnp.bitwise_and(dk, 15),
                                   1.0, 0.0).astype(jnp.float32)
                    obuf[i * 16 + k, :] = oh
                return None

            pltpu.sync_copy(obuf, acc_sp.at[idxbuf], add=True)
            return None

        plsc.subcore_barrier()

        rbase = s * _ROWS_PER_TILE
        pltpu.sync_copy(acc_sp.at[pl.ds(rbase, _ROWS_PER_TILE)], tmp)

        @functools.partial(lax.fori_loop, 0, _ROWS_PER_TILE, init_val=None)
        def _(i, _):
            outv[pl.ds(i * 16, 16)] = _rsqrt16(tmp[i, :] + 1.0)
            return None

        flat = pl.ds(rbase * 16, _ROWS_PER_TILE * 16)
        pltpu.sync_copy(outv.at[pl.ds(0, _ROWS_PER_TILE * 16)],
                        isq_sp.at[flat])
        pltpu.sync_copy(outv.at[pl.ds(0, _ROWS_PER_TILE * 16)],
                        isq_hbm.at[flat])
        plsc.subcore_barrier()

        pltpu.sync_copy(isq_sp, isq_tab)
        pltpu.sync_copy(src_hbm.at[pl.ds(base, _E_PER_TILE)], dstv)

        @functools.partial(lax.fori_loop, 0, _E_PER_TILE // 16, init_val=None)
        def _(i, _):
            sidx = dstv[pl.ds(i * 16, 16)]
            outv[pl.ds(i * 16, 16)] = plsc.load_gather(isq_tab, [sidx])
            return None

        pltpu.sync_copy(outv, isqsrc_hbm.at[pl.ds(base, _E_PER_TILE)])


def _compute_isq(src, dst):
    return pl.kernel(
        _k1_body,
        out_type=(jax.ShapeDtypeStruct((NROWS * 16,), jnp.float32),
                  jax.ShapeDtypeStruct((E,), jnp.float32)),
        mesh=plsc.VectorSubcoreMesh(core_axis_name="c", subcore_axis_name="s"),
        scratch_types=(
            pltpu.VMEM_SHARED((NROWS, 16), jnp.float32),
            pltpu.VMEM_SHARED((NROWS * 16,), jnp.float32),
            pltpu.VMEM((_E_PER_TILE,), jnp.int32),
            pltpu.VMEM((_KCH, 16), jnp.float32),
            pltpu.VMEM((_KCH,), jnp.int32),
            pltpu.VMEM((_ROWS_PER_TILE, 16), jnp.float32),
            pltpu.VMEM((NROWS * 16,), jnp.float32),
            pltpu.VMEM((_E_PER_TILE,), jnp.float32),
            pltpu.SemaphoreType.DMA,
        ),
        compiler_params=pltpu.CompilerParams(needs_layout_passes=False,
                                             use_tc_tiling_on_sc=False),
        name="sc_deg_isq",
    )(src, dst)


_NTRIPLE = (_NCHUNK - 5) // 3


def _zero_fill(zbuf):
    @functools.partial(lax.fori_loop, 0, _ZROWS, init_val=None)
    def _(i, _):
        for l in range(WC // 16):
            zbuf[i, pl.ds(l * 16, 16)] = jnp.zeros((16,), jnp.float32)
        return None


def _zero_my_slice(s, acc_sp, zbuf, semz):
    for z in range(_N_PER_TILE // _ZROWS):
        pltpu.async_copy(
            zbuf, acc_sp.at[pl.ds(s * _N_PER_TILE + z * _ZROWS, _ZROWS)],
            semz)
    for z in range(_N_PER_TILE // _ZROWS):
        pltpu.make_async_copy(
            zbuf, acc_sp.at[pl.ds(s * _N_PER_TILE + z * _ZROWS, _ZROWS)],
            semz).wait()


def _run_item(src_of, acc_sp, dstv, gbufs, semg, sems):
    def gath(c, b):
        pltpu.async_copy(src_of(c, b), gbufs[b], semg[b])

    def wait_gath(c, b):
        pltpu.make_async_copy(src_of(c, b), gbufs[b], semg[b]).wait()

    def scat(c, b):
        pltpu.async_copy(gbufs[b], acc_sp.at[dstv.at[c]], sems[b], add=True)

    def wait_scat(c, b):
        pltpu.make_async_copy(gbufs[b], acc_sp.at[dstv.at[c]], sems[b]).wait()

    for b in range(3):
        gath(b, b)

    @functools.partial(lax.fori_loop, 0, _NTRIPLE, init_val=None)
    def _(m, _):
        c0 = 3 * m
        for b in range(3):
            wait_gath(c0 + b, b)
            scat(c0 + b, b)
        for b in range(3):
            wait_scat(c0 + b, b)
            gath(c0 + 3 + b, b)
        return None

    cz = 3 * _NTRIPLE
    wait_gath(cz, 0)
    scat(cz, 0)
    wait_gath(cz + 1, 1)
    scat(cz + 1, 1)
    wait_gath(cz + 2, 2)
    scat(cz + 2, 2)
    wait_scat(cz, 0)
    gath(cz + 3, 0)
    wait_scat(cz + 1, 1)
    gath(cz + 4, 1)
    wait_gath(cz + 3, 0)
    scat(cz + 3, 0)
    wait_gath(cz + 4, 1)
    scat(cz + 4, 1)
    wait_scat(cz + 2, 2)
    wait_scat(cz + 3, 0)
    wait_scat(cz + 4, 1)


def _load_dstv(s, dst_hbm, dstv):
    pltpu.sync_copy(dst_hbm.at[s], dstv)


def _k2h_body(hs_hbm, src_hbm, dst_hbm, s_hbm,
              acc_sp, dstv, idxr, gbuf0, gbuf1, gbuf2, zbuf,
              semg0, semg1, semg2, sems0, sems1, sems2,
              semi0, semi1, semi2, semz):
    cid = lax.axis_index("c")
    s = lax.axis_index("s")
    ebase = s * _E_PER_TILE
    gbufs = (gbuf0, gbuf1, gbuf2)
    semg = (semg0, semg1, semg2)
    sems = (sems0, sems1, sems2)
    semi = (semi0, semi1, semi2)

    _load_dstv(s, dst_hbm, dstv)
    _zero_fill(zbuf)
    _zero_my_slice(s, acc_sp, zbuf, semz)
    plsc.subcore_barrier()

    def idxload(c, b):
        pltpu.async_copy(src_hbm.at[pl.ds(ebase + c * _KCH, _KCH)],
                         idxr.at[b], semi[b])

    def wait_idx(c, b):
        pltpu.make_async_copy(src_hbm.at[pl.ds(ebase + c * _KCH, _KCH)],
                              idxr.at[b], semi[b]).wait()

    for g in range(T * NCH):
        my = cid == (g % NC)

        @pl.when(my)
        def _(g=g):
            table = hs_hbm.at[g]

            def src_of(c, b):
                return table.at[idxr.at[b]]

            def gath(c, b):
                pltpu.async_copy(src_of(c, b), gbufs[b], semg[b])

            def wait_gath(c, b):
                pltpu.make_async_copy(src_of(c, b), gbufs[b], semg[b]).wait()

            def scat(c, b):
                pltpu.async_copy(gbufs[b], acc_sp.at[dstv.at[c]], sems[b],
                                 add=True)

            def wait_scat(c, b):
                pltpu.make_async_copy(gbufs[b], acc_sp.at[dstv.at[c]],
                                      sems[b]).wait()

            for b in range(3):
                idxload(b, b)
            for b in range(3):
                wait_idx(b, b)
                gath(b, b)

            @functools.partial(lax.fori_loop, 0, _NTRIPLE, init_val=None)
            def _(m, _):
                c0 = 3 * m
                for b in range(3):
                    wait_gath(c0 + b, b)
                    idxload(c0 + 3 + b, b)
                    scat(c0 + b, b)
                for b in range(3):
                    wait_idx(c0 + 3 + b, b)
                    wait_scat(c0 + b, b)
                    gath(c0 + 3 + b, b)
                return None

            cz = 3 * _NTRIPLE
            wait_gath(cz, 0)
            idxload(cz + 3, 0)
            scat(cz, 0)
            wait_gath(cz + 1, 1)
            idxload(cz + 4, 1)
            scat(cz + 1, 1)
            wait_gath(cz + 2, 2)
            scat(cz + 2, 2)
            wait_idx(cz + 3, 0)
            wait_scat(cz, 0)
            gath(cz + 3, 0)
            wait_idx(cz + 4, 1)
            wait_scat(cz + 1, 1)
            gath(cz + 4, 1)
            wait_gath(cz + 3, 0)
            scat(cz + 3, 0)
            wait_gath(cz + 4, 1)
            scat(cz + 4, 1)
            wait_scat(cz + 2, 2)
            wait_scat(cz + 3, 0)
            wait_scat(cz + 4, 1)

        plsc.subcore_barrier()

        @pl.when(my)
        def _(g=g):
            nslice = pl.ds(s * _N_PER_TILE, _N_PER_TILE)
            pltpu.sync_copy(acc_sp.at[nslice], s_hbm.at[g].at[nslice])
            _zero_my_slice(s, acc_sp, zbuf, semz)

        plsc.subcore_barrier()


def _h_segment_sums(hs_items, src, dst):
    return pl.kernel(
        _k2h_body,
        out_type=jax.ShapeDtypeStruct((T * NCH, NPAD, WC), jnp.float32),
        mesh=plsc.VectorSubcoreMesh(core_axis_name="c", subcore_axis_name="s"),
        scratch_types=(
            pltpu.VMEM_SHARED((NPAD, WC), jnp.float32),
            pltpu.VMEM((_NCHUNK, _KCH), jnp.int32),
            pltpu.VMEM((3, _KCH), jnp.int32),
            pltpu.VMEM((_KCH, WC), jnp.float32),
            pltpu.VMEM((_KCH, WC), jnp.float32),
            pltpu.VMEM((_KCH, WC), jnp.float32),
            pltpu.VMEM((_ZROWS, WC), jnp.float32),
        ) + (pltpu.SemaphoreType.DMA,) * 10,
        compiler_params=pltpu.CompilerParams(needs_layout_passes=False,
                                             use_tc_tiling_on_sc=False),
        name="sc_h_segment_sums",
    )(hs_items, src, dst.reshape(NS, _NCHUNK, _KCH))


def _k2e_body(es0_hbm, es1_hbm, dst_hbm, p0_hbm, p1_hbm,
              acc_sp, dstv, gbuf0, gbuf1, gbuf2, zbuf,
              semg0, semg1, semg2, sems0, sems1, sems2, semz):
    cid = lax.axis_index("c")
    s = lax.axis_index("s")
    ebase = s * _E_PER_TILE
    gbufs = (gbuf0, gbuf1, gbuf2)
    semg = (semg0, semg1, semg2)
    sems = (sems0, sems1, sems2)

    _load_dstv(s, dst_hbm, dstv)
    _zero_fill(zbuf)
    _zero_my_slice(s, acc_sp, zbuf, semz)
    plsc.subcore_barrier()

    for g in range(2 * NCH):
        my = cid == (g % NC)
        layer, ci = divmod(g, NCH)
        table = (es0_hbm if layer == 0 else es1_hbm).at[ci]
        out = (p0_hbm if layer == 0 else p1_hbm).at[ci]

        @pl.when(my)
        def _(table=table):
            _run_item(lambda c, b: table.at[pl.ds(ebase + c * _KCH, _KCH)],
                      acc_sp, dstv, gbufs, semg, sems)

        plsc.subcore_barrier()

        @pl.when(my)
        def _(out=out):
            nslice = pl.ds(s * _N_PER_TILE, _N_PER_TILE)
            pltpu.sync_copy(acc_sp.at[nslice], out.at[nslice])
            _zero_my_slice(s, acc_sp, zbuf, semz)

        plsc.subcore_barrier()


def _edge_segment_sums(es0, es1, dst):
    return pl.kernel(
        _k2e_body,
        out_type=(jax.ShapeDtypeStruct((NCH, NPAD, WC), jnp.float32),
                  jax.ShapeDtypeStruct((NCH, NPAD, WC), jnp.float32)),
        mesh=plsc.VectorSubcoreMesh(core_axis_name="c", subcore_axis_name="s"),
        scratch_types=(
            pltpu.VMEM_SHARED((NPAD, WC), jnp.float32),
            pltpu.VMEM((_NCHUNK, _KCH), jnp.int32),
            pltpu.VMEM((_KCH, WC), jnp.float32),
            pltpu.VMEM((_KCH, WC), jnp.float32),
            pltpu.VMEM((_KCH, WC), jnp.float32),
            pltpu.VMEM((_ZROWS, WC), jnp.float32),
        ) + (pltpu.SemaphoreType.DMA,) * 7,
        compiler_params=pltpu.CompilerParams(needs_layout_passes=False,
                                             use_tc_tiling_on_sc=False),
        name="sc_edge_segment_sums",
    )(es0, es1, dst.reshape(NS, _NCHUNK, _KCH))


_NT = 1000
_ET = 2000


def _k3a_body(x_ref, w_ref, isq_ref, hs_ref):
    y = jnp.dot(x_ref[0], w_ref[...], preferred_element_type=jnp.float32)
    y = y * isq_ref[...]
    for ci in range(NCH):
        hs_ref[0, ci] = y[:, ci * WC:(ci + 1) * WC]


def _node_proj(x, wcat, isq_col):
    return pl.pallas_call(
        _k3a_body,
        grid=(T, N // _NT),
        in_specs=[
            pl.BlockSpec((1, _NT, HID), lambda t, n: (t, n, 0)),
            pl.BlockSpec((HID, 2 * HID), lambda t, n: (0, 0)),
            pl.BlockSpec((_NT, 1), lambda t, n: (n, 0)),
        ],
        out_specs=pl.BlockSpec((1, NCH, _NT, WC), lambda t, n: (t, 0, n, 0)),
        out_shape=jax.ShapeDtypeStruct((T, NCH, N, WC), jnp.float32),
    )(x, wcat, isq_col)


def _k3b_body(ea_ref, w_ref, isqs_ref, es_ref):
    y = jnp.dot(ea_ref[...], w_ref[...], preferred_element_type=jnp.float32)
    y = jnp.maximum(y, 0.0) * isqs_ref[...]
    for ci in range(NCH):
        es_ref[ci] = y[:, ci * WC:(ci + 1) * WC]


def _edge_proj(edge_attr, wecat, isq_src_col):
    return pl.pallas_call(
        _k3b_body,
        grid=(E // _ET,),
        in_specs=[
            pl.BlockSpec((_ET, F_EDGE), lambda e: (e, 0)),
            pl.BlockSpec((F_EDGE, 2 * HID), lambda e: (0, 0)),
            pl.BlockSpec((_ET, 1), lambda e: (e, 0)),
        ],
        out_specs=pl.BlockSpec((NCH, _ET, WC), lambda e: (0, e, 0)),
        out_shape=jax.ShapeDtypeStruct((NCH, E, WC), jnp.float32),
    )(edge_attr, wecat, isq_src_col)


def _k5_body(s_ref, p_ref, hs_ref, isq_ref, b_ref, out_ref, *, apply_relu):
    isq = isq_ref[...]

    def gcn(ci):
        return (isq * (s_ref[0, ci] + p_ref[ci] + hs_ref[0, ci])
                + b_ref[0:1, ci * WC:(ci + 1) * WC])

    half = NCH // 2
    for ci in range(half):
        o = (1.0 - jax.nn.sigmoid(gcn(ci))) * jnp.tanh(gcn(ci + half))
        if apply_relu:
            o = jnp.maximum(o, 0.0)
        out_ref[0, :, ci * WC:(ci + 1) * WC] = o


def _gate_combine(S4, P, hs, isq_col, bcat, apply_relu):
    return pl.pallas_call(
        functools.partial(_k5_body, apply_relu=apply_relu),
        grid=(T, N // _NT),
        in_specs=[
            pl.BlockSpec((1, NCH, _NT, WC), lambda t, n: (t, 0, n, 0)),
            pl.BlockSpec((NCH, _NT, WC), lambda t, n: (0, n, 0)),
            pl.BlockSpec((1, NCH, _NT, WC), lambda t, n: (t, 0, n, 0)),
            pl.BlockSpec((_NT, 1), lambda t, n: (n, 0)),
            pl.BlockSpec((1, 2 * HID), lambda t, n: (0, 0)),
        ],
        out_specs=pl.BlockSpec((1, _NT, HID), lambda t, n: (t, n, 0)),
        out_shape=jax.ShapeDtypeStruct((T, N, HID), jnp.float32),
    )(S4, P, hs, isq_col, bcat)


def _layer(x_in, src, dst, isq_col, P, wcat, bcat, apply_relu):
    hs = _node_proj(x_in, wcat, isq_col)
    S = _h_segment_sums(hs.reshape(T * NCH, N, WC), src, dst)
    return _gate_combine(S.reshape(T, NCH, NPAD, WC), P, hs, isq_col, bcat,
                         apply_relu)


def kernel(x, edge_index, edge_attr, g0Wn, g0We, g0b, u0Wn, u0We, u0b,
           g1Wn, g1We, g1b, u1Wn, u1We, u1b):
    src = edge_index[0]
    dst = edge_index[1]

    isq_flat, isq_src = _compute_isq(src, dst)
    isq_col = isq_flat[:N].reshape(N, 1)
    isq_src_col = isq_src.reshape(E, 1)

    wcat0 = jnp.concatenate([g0Wn[:HID, HID:], u0Wn[:HID, :]], axis=1)
    wcat1 = jnp.concatenate([g1Wn[:HID, HID:], u1Wn[:HID, :]], axis=1)
    bcat0 = jnp.concatenate([g0b[HID:], u0b]).reshape(1, 2 * HID)
    bcat1 = jnp.concatenate([g1b[HID:], u1b]).reshape(1, 2 * HID)
    wecat0 = jnp.concatenate([g0We[:, HID:], u0We], axis=1)
    wecat1 = jnp.concatenate([g1We[:, HID:], u1We], axis=1)

    es0 = _edge_proj(edge_attr, wecat0, isq_src_col)
    es1 = _edge_proj(edge_attr, wecat1, isq_src_col)
    P0, P1 = _edge_segment_sums(es0, es1, dst)

    h0 = _layer(x, src, dst, isq_col, P0, wcat0, bcat0, apply_relu=True)
    h1 = _layer(h0, src, dst, isq_col, P1, wcat1, bcat1, apply_relu=False)
    return h1
